# baseline (speedup 1.0000x reference)
import functools

import jax
import jax.numpy as jnp
from jax import lax
from jax.experimental import pallas as pl
from jax.experimental.pallas import tpu as pltpu
from jax.experimental.pallas import tpu_sc as plsc

CHUNK = 128


@functools.lru_cache(maxsize=None)
def _build(v, d, b):
    info = plsc.get_sparse_core_info()
    nc, ns = info.num_cores, info.num_subcores
    nw = nc * ns
    assert b % (nw * CHUNK) == 0
    n_chunks = b // (nw * CHUNK)

    mesh = plsc.VectorSubcoreMesh(core_axis_name="c", subcore_axis_name="s")

    @functools.partial(
        pl.kernel,
        mesh=mesh,
        compiler_params=pltpu.CompilerParams(use_tc_tiling_on_sc=False),
        out_type=[
            jax.ShapeDtypeStruct((b // CHUNK, CHUNK, d), jnp.float32),
            jax.ShapeDtypeStruct((b // CHUNK, CHUNK, d), jnp.float32),
        ],
        scratch_types=[
            pltpu.VMEM((n_chunks, CHUNK), jnp.int32),
            pltpu.VMEM((n_chunks, CHUNK, d), jnp.float32),
            pltpu.VMEM((n_chunks, CHUNK, d), jnp.float32),
            pltpu.SemaphoreType.DMA,
        ],
    )
    def k(funcs_hbm, measures_hbm, ranks_hbm, f_out, m_out, idx_v, frows, mrows, sem):
        wid = lax.axis_index("s") * nc + lax.axis_index("c")
        base = wid * n_chunks
        pltpu.sync_copy(ranks_hbm.at[pl.ds(base, n_chunks)], idx_v)
        copies = []
        for j in range(n_chunks):
            copies.append(pltpu.async_copy(funcs_hbm.at[idx_v.at[j]], frows.at[j], sem))
            copies.append(pltpu.async_copy(measures_hbm.at[idx_v.at[j]], mrows.at[j], sem))
        for c in copies:
            c.wait()
        pltpu.sync_copy(frows, f_out.at[pl.ds(base, n_chunks)])
        pltpu.sync_copy(mrows, m_out.at[pl.ds(base, n_chunks)])

    return k


def kernel(funcs, measures, ranks):
    v, d = funcs.shape
    b = ranks.shape[0]
    ranks2d = ranks.reshape(b // CHUNK, CHUNK)
    f_sel, m_sel = _build(v, d, b)(funcs, measures, ranks2d)
    return (f_sel.reshape(b, d), m_sel.reshape(b, d))
